# baseline (speedup 1.0000x reference)
import functools

import jax
import jax.numpy as jnp
from jax import lax
from jax.experimental import pallas as pl
from jax.experimental.pallas import tpu as pltpu
from jax.experimental.pallas import tpu_sc as plsc

LPAD = 56
LREAL = 50
LANES = 16


def _accum_row(rows_v, zbuf_v, j):
  nd = rows_v.shape[1] // LANES

  def body(l, acc):
    return tuple(
        acc[d] + rows_v[l, pl.ds(d * LANES, LANES)] for d in range(nd)
    )

  init = tuple(rows_v[0, pl.ds(d * LANES, LANES)] for d in range(nd))
  acc = lax.fori_loop(1, LREAL, body, init, unroll=2)
  for d in range(nd):
    zbuf_v[j, pl.ds(d * LANES, LANES)] = acc[d]


def _sc_embed(xflat, table):
  B = xflat.shape[0] // LPAD
  D = table.shape[1]
  mesh = plsc.VectorSubcoreMesh(core_axis_name="c", subcore_axis_name="s")
  NW = mesh.num_cores * mesh.num_subcores
  bpw = B // NW
  steps = bpw // 2

  @functools.partial(
      pl.kernel,
      out_type=jax.ShapeDtypeStruct((B, D), jnp.float32),
      mesh=mesh,
      scratch_types=[
          pltpu.VMEM((bpw * LPAD,), jnp.int32),
          pltpu.VMEM((LPAD, D), jnp.float32),
          pltpu.VMEM((LPAD, D), jnp.float32),
          pltpu.VMEM((2, D), jnp.float32),
          pltpu.SemaphoreType.DMA,
          pltpu.SemaphoreType.DMA,
      ],
  )
  def k(xflat_hbm, table_hbm, out_hbm, idx_v, buf0, buf1, zbuf_v, sem0, sem1):
    wid = lax.axis_index("s") * mesh.num_cores + lax.axis_index("c")
    base = wid * bpw

    pltpu.sync_copy(xflat_hbm.at[pl.ds(base * LPAD, bpw * LPAD)], idx_v)

    pltpu.async_copy(table_hbm.at[idx_v.at[pl.ds(0, LPAD)]], buf0, sem0)

    def step(s, carry):
      r0 = 2 * s
      pltpu.async_copy(
          table_hbm.at[idx_v.at[pl.ds((r0 + 1) * LPAD, LPAD)]], buf1, sem1)
      pltpu.make_async_copy(
          table_hbm.at[idx_v.at[pl.ds(r0 * LPAD, LPAD)]], buf0, sem0).wait()
      _accum_row(buf0, zbuf_v, 0)

      @pl.when(s < steps - 1)
      def _():
        pltpu.async_copy(
            table_hbm.at[idx_v.at[pl.ds((r0 + 2) * LPAD, LPAD)]], buf0, sem0)

      pltpu.make_async_copy(
          table_hbm.at[idx_v.at[pl.ds((r0 + 1) * LPAD, LPAD)]], buf1, sem1
      ).wait()
      _accum_row(buf1, zbuf_v, 1)
      pltpu.sync_copy(zbuf_v, out_hbm.at[pl.ds(base + r0, 2)])
      return carry

    lax.fori_loop(0, steps, step, 0)

  return k(xflat, table)


def _mlp_body(z_ref, w1_ref, b1_ref, w2_ref, b2_ref, w3_ref, b3_ref,
              turn_ref, o_ref):
  z = z_ref[...]
  h = lax.dot_general(z, w1_ref[...], (((1,), (1,)), ((), ())),
                      preferred_element_type=jnp.float32)
  h = jnp.clip(h + b1_ref[...], 0.0, 1.0)
  h = lax.dot_general(h, w2_ref[...], (((1,), (1,)), ((), ())),
                      preferred_element_type=jnp.float32)
  h = jnp.clip(h + b2_ref[...], 0.0, 1.0)
  o = jnp.sum(h * w3_ref[...], axis=1, keepdims=True) + b3_ref[...]
  o_ref[...] = o * turn_ref[...]


def _tc_mlp(z, W1, b1, W2, b2, W3, b3, turn):
  B, D = z.shape
  BT = 2048
  grid = B // BT
  return pl.pallas_call(
      _mlp_body,
      grid=(grid,),
      in_specs=[
          pl.BlockSpec((BT, D), lambda i: (i, 0)),
          pl.BlockSpec(W1.shape, lambda i: (0, 0)),
          pl.BlockSpec(b1.shape, lambda i: (0, 0)),
          pl.BlockSpec(W2.shape, lambda i: (0, 0)),
          pl.BlockSpec(b2.shape, lambda i: (0, 0)),
          pl.BlockSpec(W3.shape, lambda i: (0, 0)),
          pl.BlockSpec(b3.shape, lambda i: (0, 0)),
          pl.BlockSpec((BT, 1), lambda i: (i, 0)),
      ],
      out_specs=pl.BlockSpec((BT, 1), lambda i: (i, 0)),
      out_shape=jax.ShapeDtypeStruct((B, 1), jnp.float32),
  )(z, W1, b1, W2, b2, W3, b3, turn)


def kernel(x, turn, table, W1, b1, W2, b2, W3, b3):
  B, L = x.shape
  xpad = jnp.pad(x.astype(jnp.int32), ((0, 0), (0, LPAD - L)))
  z = _sc_embed(xpad.reshape(-1), table)
  return _tc_mlp(z, W1, b1.reshape(1, -1), W2, b2.reshape(1, -1),
                 W3, b3.reshape(1, 1), turn)


# speedup vs baseline: 11.1437x; 11.1437x over previous
import functools

import jax
import jax.numpy as jnp
from jax import lax
from jax.experimental import pallas as pl
from jax.experimental.pallas import tpu as pltpu
from jax.experimental.pallas import tpu_sc as plsc

LPAD = 56
LREAL = 50
LANES = 16
NBUF = 4


def _accum_row(rows_v, zbuf_v, j):
  nd = rows_v.shape[1] // LANES

  def body(l, acc):
    return tuple(
        acc[d] + rows_v[l, pl.ds(d * LANES, LANES)] for d in range(nd)
    )

  init = tuple(rows_v[0, pl.ds(d * LANES, LANES)] for d in range(nd))
  acc = lax.fori_loop(1, LREAL, body, init, unroll=7)
  for d in range(nd):
    zbuf_v[j, pl.ds(d * LANES, LANES)] = acc[d]


def _sc_embed(xflat, table):
  B = xflat.shape[0] // LPAD
  D = table.shape[1]
  mesh = plsc.VectorSubcoreMesh(core_axis_name="c", subcore_axis_name="s")
  NW = mesh.num_cores * mesh.num_subcores
  bpw = B // NW
  steps = bpw // NBUF

  @functools.partial(
      pl.kernel,
      out_type=jax.ShapeDtypeStruct((B, D), jnp.float32),
      mesh=mesh,
      scratch_types=[
          pltpu.VMEM((bpw * LPAD,), jnp.int32),
          pltpu.VMEM((NBUF, LPAD, D), jnp.float32),
          pltpu.VMEM((NBUF, D), jnp.float32),
          [pltpu.SemaphoreType.DMA] * NBUF,
      ],
  )
  def k(xflat_hbm, table_hbm, out_hbm, idx_v, bufs, zbuf_v, sems):
    wid = lax.axis_index("s") * mesh.num_cores + lax.axis_index("c")
    base = wid * bpw

    def gather(r, b):
      pltpu.async_copy(
          table_hbm.at[idx_v.at[pl.ds(r * LPAD, LPAD)]], bufs.at[b], sems[b])

    def gather_wait(r, b):
      pltpu.make_async_copy(
          table_hbm.at[idx_v.at[pl.ds(r * LPAD, LPAD)]], bufs.at[b], sems[b]
      ).wait()

    pltpu.sync_copy(xflat_hbm.at[pl.ds(base * LPAD, bpw * LPAD)], idx_v)

    for b in range(NBUF):
      gather(b, b)

    def step(s, carry):
      r0 = NBUF * s
      for b in range(NBUF):
        gather_wait(r0 + b, b)
        _accum_row(bufs.at[b], zbuf_v, b)

        @pl.when(s < steps - 1)
        def _():
          gather(r0 + b + NBUF, b)

      pltpu.sync_copy(zbuf_v, out_hbm.at[pl.ds(base + r0, NBUF)])
      return carry

    lax.fori_loop(0, steps, step, 0)

  return k(xflat, table)


def _mlp_body(z_ref, w1_ref, b1_ref, w2_ref, b2_ref, w3_ref, b3_ref,
              turn_ref, o_ref):
  z = z_ref[...]
  h = lax.dot_general(z, w1_ref[...], (((1,), (1,)), ((), ())),
                      preferred_element_type=jnp.float32)
  h = jnp.clip(h + b1_ref[...], 0.0, 1.0)
  h = lax.dot_general(h, w2_ref[...], (((1,), (1,)), ((), ())),
                      preferred_element_type=jnp.float32)
  h = jnp.clip(h + b2_ref[...], 0.0, 1.0)
  o = jnp.sum(h * w3_ref[...], axis=1, keepdims=True) + b3_ref[...]
  o_ref[...] = o * turn_ref[...]


def _tc_mlp(z, W1, b1, W2, b2, W3, b3, turn):
  B, D = z.shape
  BT = 2048
  grid = B // BT
  return pl.pallas_call(
      _mlp_body,
      grid=(grid,),
      in_specs=[
          pl.BlockSpec((BT, D), lambda i: (i, 0)),
          pl.BlockSpec(W1.shape, lambda i: (0, 0)),
          pl.BlockSpec(b1.shape, lambda i: (0, 0)),
          pl.BlockSpec(W2.shape, lambda i: (0, 0)),
          pl.BlockSpec(b2.shape, lambda i: (0, 0)),
          pl.BlockSpec(W3.shape, lambda i: (0, 0)),
          pl.BlockSpec(b3.shape, lambda i: (0, 0)),
          pl.BlockSpec((BT, 1), lambda i: (i, 0)),
      ],
      out_specs=pl.BlockSpec((BT, 1), lambda i: (i, 0)),
      out_shape=jax.ShapeDtypeStruct((B, 1), jnp.float32),
  )(z, W1, b1, W2, b2, W3, b3, turn)


def kernel(x, turn, table, W1, b1, W2, b2, W3, b3):
  B, L = x.shape
  xi = x.astype(jnp.int32)
  xpad = jnp.concatenate([xi, xi[:, : LPAD - L]], axis=1)
  z = _sc_embed(xpad.reshape(-1), table)
  return _tc_mlp(z, W1, b1.reshape(1, -1), W2, b2.reshape(1, -1),
                 W3, b3.reshape(1, 1), turn)
